# baseline (speedup 1.0000x reference)
import functools

import jax
import jax.numpy as jnp
import numpy as np
from jax import lax
from jax.experimental import pallas as pl
from jax.experimental.pallas import tpu as pltpu
from jax.experimental.pallas import tpu_sc as plsc

BATCH = 16384
NUM_FIELDS = 26
NUM_WORKERS = 32
ROWS_W = BATCH // NUM_WORKERS
PER_W = ROWS_W * NUM_FIELDS
N_CHUNK = 8
CHUNK = PER_W // N_CHUNK
VEC = 16
N_GROUPS = PER_W // VEC
GROUPS_PER_CHUNK = N_GROUPS // N_CHUNK


def _make_kernel():
    mesh = plsc.VectorSubcoreMesh(core_axis_name="c", subcore_axis_name="s")

    @functools.partial(
        pl.kernel,
        mesh=mesh,
        out_type=jax.ShapeDtypeStruct((BATCH, NUM_FIELDS), jnp.int32),
        compiler_params=pltpu.CompilerParams(needs_layout_passes=False),
        scratch_types=[
            pltpu.VMEM((ROWS_W, NUM_FIELDS), jnp.int32),
            pltpu.VMEM((PER_W,), jnp.int32),
            pltpu.VMEM((PER_W,), jnp.int32),
            pltpu.VMEM((PER_W,), jnp.int32),
            pltpu.VMEM((PER_W,), jnp.int32),
            pltpu.SemaphoreType.DMA,
            pltpu.SemaphoreType.DMA,
        ]
        + [pltpu.SemaphoreType.DMA for _ in range(N_CHUNK)],
    )
    def k(idx_hbm, table_hbm, rmap_hbm, cmap_hbm, out_hbm,
          buf2d_v, idx_v, rows_v, rmap_v, cmap_v, sem_i, sem_m, *gsems):
        wid = lax.axis_index("s") * 2 + lax.axis_index("c")
        rbase = wid * ROWS_W
        st1 = pltpu.async_copy(
            idx_hbm.at[pl.ds(rbase, ROWS_W), :], buf2d_v, sem_i)
        st2 = pltpu.async_copy(rmap_hbm, rmap_v, sem_m)
        st3 = pltpu.async_copy(cmap_hbm, cmap_v, sem_m)
        st2.wait()
        st3.wait()
        st1.wait()

        def flatten_body(i, _):
            p0 = i * VEC
            rv = rmap_v[pl.ds(p0, VEC)]
            cv = cmap_v[pl.ds(p0, VEC)]
            idx_v[pl.ds(p0, VEC)] = plsc.load_gather(buf2d_v, (rv, cv))
            return _

        lax.fori_loop(0, N_GROUPS, flatten_body, None, unroll=8)

        gathers = [
            pltpu.async_copy(
                table_hbm.at[idx_v.at[pl.ds(j * CHUNK, CHUNK)]],
                rows_v.at[pl.ds(j * CHUNK, CHUNK)],
                gsems[j],
            )
            for j in range(N_CHUNK)
        ]

        def unflatten_body(i, _):
            p0 = i * VEC
            rv = rmap_v[pl.ds(p0, VEC)]
            cv = cmap_v[pl.ds(p0, VEC)]
            plsc.store_scatter(buf2d_v, (rv, cv), rows_v[pl.ds(p0, VEC)])
            return _

        for j in range(N_CHUNK):
            gathers[j].wait()
            lax.fori_loop(
                j * GROUPS_PER_CHUNK, (j + 1) * GROUPS_PER_CHUNK,
                unflatten_body, None, unroll=8)

        pltpu.sync_copy(buf2d_v, out_hbm.at[pl.ds(rbase, ROWS_W), :])

    return k


_gather_kernel = _make_kernel()

_ROWMAP = np.arange(PER_W, dtype=np.int32) // NUM_FIELDS
_COLMAP = np.arange(PER_W, dtype=np.int32) % NUM_FIELDS


def kernel(inputs, table_vals):
    return _gather_kernel(inputs, table_vals, _ROWMAP, _COLMAP)


# speedup vs baseline: 1.8227x; 1.8227x over previous
import functools

import jax
import jax.numpy as jnp
from jax import lax
from jax.experimental import pallas as pl
from jax.experimental.pallas import tpu as pltpu
from jax.experimental.pallas import tpu_sc as plsc

BATCH = 16384
NUM_FIELDS = 26
NUM_WORKERS = 32
COLS_W = BATCH // NUM_WORKERS
PER_W = COLS_W * NUM_FIELDS
N_CHUNK = 13
CHUNK = PER_W // N_CHUNK
ROWS_PER_CHUNK = NUM_FIELDS // N_CHUNK


def _make_kernel():
    mesh = plsc.VectorSubcoreMesh(core_axis_name="c", subcore_axis_name="s")

    @functools.partial(
        pl.kernel,
        mesh=mesh,
        out_type=jax.ShapeDtypeStruct((NUM_FIELDS, BATCH), jnp.int32),
        compiler_params=pltpu.CompilerParams(needs_layout_passes=False),
        scratch_types=[
            pltpu.VMEM((PER_W,), jnp.int32),
            pltpu.VMEM((PER_W,), jnp.int32),
            pltpu.SemaphoreType.DMA,
            pltpu.SemaphoreType.DMA,
        ]
        + [pltpu.SemaphoreType.DMA for _ in range(N_CHUNK)],
    )
    def k(idx_hbm, table_hbm, out_hbm, idx_v, rows_v, sem_i, sem_o, *gsems):
        wid = lax.axis_index("s") * 2 + lax.axis_index("c")
        cbase = wid * COLS_W
        stages = [
            pltpu.async_copy(
                idx_hbm.at[r, pl.ds(cbase, COLS_W)],
                idx_v.at[pl.ds(r * COLS_W, COLS_W)],
                sem_i,
            )
            for r in range(NUM_FIELDS)
        ]
        for st in stages:
            st.wait()

        gathers = [
            pltpu.async_copy(
                table_hbm.at[idx_v.at[pl.ds(j * CHUNK, CHUNK)]],
                rows_v.at[pl.ds(j * CHUNK, CHUNK)],
                gsems[j],
            )
            for j in range(N_CHUNK)
        ]
        outs = []
        for j in range(N_CHUNK):
            gathers[j].wait()
            for r in range(j * ROWS_PER_CHUNK, (j + 1) * ROWS_PER_CHUNK):
                outs.append(
                    pltpu.async_copy(
                        rows_v.at[pl.ds(r * COLS_W, COLS_W)],
                        out_hbm.at[r, pl.ds(cbase, COLS_W)],
                        sem_o,
                    )
                )
        for c in outs:
            c.wait()

    return k


_gather_kernel = _make_kernel()


def kernel(inputs, table_vals):
    out_t = _gather_kernel(inputs.T, table_vals)
    return out_t.T
